# baseline (speedup 1.0000x reference)
import functools

import jax
import jax.numpy as jnp
from jax import lax
from jax.experimental import pallas as pl
from jax.experimental.pallas import tpu as pltpu
from jax.experimental.pallas import tpu_sc as plsc

VOCAB = 30522
D = 768
NCLS = 2
B = 4096
L = 200

BLK = 3072
VP = 30720
BIAS_COL = 30528

NC = 2
NS = 16
NW = NC * NS
SEQ_PER_W = B // NW
GROUPS = SEQ_PER_W // 16


def _tw_body(w_ref, tbl_ref, b_ref, out_ref):
    out_ref[...] = lax.dot_general(
        w_ref[...], tbl_ref[...],
        dimension_numbers=(((1,), (1,)), ((), ())),
        preferred_element_type=jnp.float32,
    )

    @pl.when(pl.program_id(0) == BIAS_COL // BLK)
    def _():
        out_ref[:, pl.ds(BIAS_COL % BLK, 1)] = b_ref[...]


def _compute_tw(table, W, b2):
    return pl.pallas_call(
        _tw_body,
        grid=(VP // BLK,),
        in_specs=[
            pl.BlockSpec((NCLS, D), lambda i: (0, 0)),
            pl.BlockSpec((BLK, D), lambda i: (i, 0)),
            pl.BlockSpec((NCLS, 1), lambda i: (0, 0)),
        ],
        out_specs=pl.BlockSpec((NCLS, BLK), lambda i: (0, i)),
        out_shape=jax.ShapeDtypeStruct((NCLS, VP), jnp.float32),
    )(W, table, b2)


def _sc_kernel(tw_hbm, ids_hbm, out_hbm, tw_v, ids_v, out_v, sem_a, sem_b):
    wid = lax.axis_index("s") * NC + lax.axis_index("c")
    base = wid * SEQ_PER_W

    cp_tw = pltpu.async_copy(tw_hbm, tw_v, sem_a)
    cp_ids = pltpu.async_copy(
        ids_hbm.at[pl.ds(base * L, SEQ_PER_W * L)], ids_v, sem_b)
    cp_tw.wait()
    cp_ids.wait()

    lane = lax.iota(jnp.int32, 16)
    stride = lane * L
    zero = jnp.zeros((16,), jnp.float32)

    def body(t, accs):
        new = []
        for g in range(GROUPS):
            addr = stride + (g * 16 * L + t)
            idx = plsc.load_gather(ids_v, [addr])
            v0 = plsc.load_gather(tw_v, [idx])
            v1 = plsc.load_gather(tw_v, [idx + VP])
            new.append(accs[2 * g] + v0)
            new.append(accs[2 * g + 1] + v1)
        return tuple(new)

    accs = lax.fori_loop(0, L, body, (zero,) * (2 * GROUPS))

    inv_l = jnp.float32(1.0 / L)
    b0 = tw_v[pl.ds(BIAS_COL, 16)][0]
    b1 = tw_v[pl.ds(VP + BIAS_COL, 16)][0]
    col0 = jnp.zeros((16,), jnp.int32)
    col1 = jnp.ones((16,), jnp.int32)
    for g in range(GROUPS):
        rows = lane + g * 16
        plsc.store_scatter(out_v, [rows, col0], accs[2 * g] * inv_l + b0)
        plsc.store_scatter(out_v, [rows, col1], accs[2 * g + 1] * inv_l + b1)

    pltpu.sync_copy(out_v, out_hbm.at[pl.ds(base, SEQ_PER_W)])


def _pool_logits(tw_flat, ids_flat):
    mesh = plsc.VectorSubcoreMesh(core_axis_name="c", subcore_axis_name="s")
    f = functools.partial(
        pl.kernel,
        mesh=mesh,
        out_type=jax.ShapeDtypeStruct((B, NCLS), jnp.float32),
        scratch_types=[
            pltpu.VMEM((2 * VP,), jnp.float32),
            pltpu.VMEM((SEQ_PER_W * L,), jnp.int32),
            pltpu.VMEM((SEQ_PER_W, NCLS), jnp.float32),
            pltpu.SemaphoreType.DMA,
            pltpu.SemaphoreType.DMA,
        ],
        compiler_params=pltpu.CompilerParams(needs_layout_passes=False),
    )(_sc_kernel)
    return f(tw_flat, ids_flat)


def kernel(input_ids, table, W, b):
    b2 = b.astype(jnp.float32).reshape(NCLS, 1)
    tw = _compute_tw(table, W, b2)
    tw_flat = tw.reshape(2 * VP)
    ids_flat = input_ids.astype(jnp.int32).reshape(B * L)
    return _pool_logits(tw_flat, ids_flat)
